# baseline (speedup 1.0000x reference)
import jax
import jax.numpy as jnp
from jax import lax
from jax.experimental import pallas as pl
from jax.experimental.pallas import tpu as pltpu
from jax.experimental.pallas import tpu_sc as plsc

NC = 2
NS = 16
NW = NC * NS
D = 64
CH = 128
L = 16


def _body(x_hbm, w_hbm, out_hbm, jv, ov, rawA, rawB, tbA, tbB, gA, gB, sA, sB):
    n_b1 = x_hbm.shape[0]
    wid = lax.axis_index("s") * NC + lax.axis_index("c")
    b0 = wid * CH

    pltpu.sync_copy(x_hbm.at[:, pl.ds(b0, CH)], jv)

    @pl.loop(0, n_b1)
    def _(b1):
        for g in range(CH // L):
            sl = pl.ds(g * L, L)
            v = jv[b1, sl]
            ov[b1, sl] = lax.shift_left(lax.bitwise_and(v, 1), 6)
            jv[b1, sl] = lax.shift_right_logical(v, 1)

    def fire_gather(b1, raw, sem):
        pltpu.async_copy(w_hbm.at[jv.at[b1]], raw, sem)

    def wait_gather(raw, sem):
        pltpu.make_async_copy(w_hbm.at[jv.at[0]], raw, sem).wait()

    def transpose(b1, raw, tb):
        for g in range(CH // L):
            offv = ov[b1, pl.ds(g * L, L)]
            rows = lax.iota(jnp.int32, L) + g * L
            for d in range(D):
                tb[d, g * L:(g + 1) * L] = plsc.load_gather(
                    raw, [rows, offv + d])

    def fire_store(b1, tb, sem):
        pltpu.async_copy(tb, out_hbm.at[b1, :, pl.ds(b0, CH)], sem)

    def wait_store(tb, sem):
        pltpu.make_async_copy(tb, out_hbm.at[0, :, pl.ds(b0, CH)], sem).wait()

    fire_gather(0, rawA, gA)

    @pl.loop(0, n_b1 // 2)
    def _(i):
        b1a = 2 * i
        b1b = 2 * i + 1
        wait_gather(rawA, gA)
        fire_gather(b1b, rawB, gB)

        @pl.when(i > 0)
        def _():
            wait_store(tbA, sA)

        transpose(b1a, rawA, tbA)
        fire_store(b1a, tbA, sA)

        wait_gather(rawB, gB)

        @pl.when(b1b + 1 < n_b1)
        def _():
            fire_gather(b1b + 1, rawA, gA)

        @pl.when(i > 0)
        def _():
            wait_store(tbB, sB)

        transpose(b1b, rawB, tbB)
        fire_store(b1b, tbB, sB)

    wait_store(tbA, sA)
    wait_store(tbB, sB)


def kernel(x, weight):
    b0n, b1n = x.shape
    nvoc, d = weight.shape
    assert d == D and b0n == NW * CH
    xT = x.T.astype(jnp.int32)
    w2 = weight.reshape(nvoc // 2, 2 * D)

    mesh = plsc.VectorSubcoreMesh(core_axis_name="c", subcore_axis_name="s")
    outT = pl.kernel(
        _body,
        out_type=jax.ShapeDtypeStruct((b1n, D, b0n), jnp.float32),
        mesh=mesh,
        scratch_types=[
            pltpu.VMEM((b1n, CH), jnp.int32),
            pltpu.VMEM((b1n, CH), jnp.int32),
            pltpu.VMEM((CH, 2 * D), jnp.float32),
            pltpu.VMEM((CH, 2 * D), jnp.float32),
            pltpu.VMEM((D, CH), jnp.float32),
            pltpu.VMEM((D, CH), jnp.float32),
            pltpu.SemaphoreType.DMA,
            pltpu.SemaphoreType.DMA,
            pltpu.SemaphoreType.DMA,
            pltpu.SemaphoreType.DMA,
        ],
        compiler_params=pltpu.CompilerParams(
            use_tc_tiling_on_sc=True, needs_layout_passes=False),
    )(xT, w2)
    return jnp.transpose(outT, (2, 0, 1))


# speedup vs baseline: 1.1946x; 1.1946x over previous
import jax
import jax.numpy as jnp
from jax import lax
from jax.experimental import pallas as pl
from jax.experimental.pallas import tpu as pltpu
from jax.experimental.pallas import tpu_sc as plsc

NC = 2
NS = 16
NW = NC * NS
D = 64
CH = 128
L = 16


def _body(x_hbm, w_hbm, out_hbm, jv, ov, rawA, rawB, tbA, tbB, gA, gB, sA, sB):
    n_b1 = x_hbm.shape[0]
    wid = lax.axis_index("s") * NC + lax.axis_index("c")
    b0 = wid * CH

    pltpu.sync_copy(x_hbm.at[:, pl.ds(b0, CH)], jv)

    @pl.loop(0, n_b1)
    def _(b1):
        for g in range(CH // L):
            sl = pl.ds(g * L, L)
            v = jv[b1, sl]
            ov[b1, sl] = lax.shift_left(lax.bitwise_and(v, 1), 6)
            jv[b1, sl] = lax.shift_right_logical(v, 1)

    def fire_gather(b1, raw, sem):
        pltpu.async_copy(w_hbm.at[jv.at[b1]], raw, sem)

    def wait_gather(raw, sem):
        pltpu.make_async_copy(w_hbm.at[jv.at[0]], raw, sem).wait()

    rowv = [lax.iota(jnp.int32, L) + dd * L for dd in range(D // L)]

    def transpose(b1, raw, tb):
        @pl.loop(0, CH // L)
        def _(g):
            offv = ov[b1, pl.ds(g * L, L)]
            for k in range(L):
                t = g * L + k
                off = offv[k]
                colv = lax.broadcast(t, (L,))
                for dd in range(D // L):
                    vals = raw[t, pl.ds(off + dd * L, L)]
                    plsc.store_scatter(tb, [rowv[dd], colv], vals)

    def fire_store(b1, tb, sem):
        pltpu.async_copy(
            tb.at[:, pl.ds(0, CH)], out_hbm.at[b1, :, pl.ds(b0, CH)], sem)

    def wait_store(tb, sem):
        pltpu.make_async_copy(
            tb.at[:, pl.ds(0, CH)], out_hbm.at[0, :, pl.ds(b0, CH)], sem
        ).wait()

    fire_gather(0, rawA, gA)

    @pl.loop(0, n_b1 // 2)
    def _(i):
        b1a = 2 * i
        b1b = 2 * i + 1
        wait_gather(rawA, gA)
        fire_gather(b1b, rawB, gB)

        @pl.when(i > 0)
        def _():
            wait_store(tbA, sA)

        transpose(b1a, rawA, tbA)
        fire_store(b1a, tbA, sA)

        wait_gather(rawB, gB)

        @pl.when(b1b + 1 < n_b1)
        def _():
            fire_gather(b1b + 1, rawA, gA)

        @pl.when(i > 0)
        def _():
            wait_store(tbB, sB)

        transpose(b1b, rawB, tbB)
        fire_store(b1b, tbB, sB)

    wait_store(tbA, sA)
    wait_store(tbB, sB)


def kernel(x, weight):
    b0n, b1n = x.shape
    nvoc, d = weight.shape
    assert d == D and b0n == NW * CH
    xT = x.T.astype(jnp.int32)
    w2 = weight.reshape(nvoc // 2, 2 * D)

    mesh = plsc.VectorSubcoreMesh(core_axis_name="c", subcore_axis_name="s")
    outT = pl.kernel(
        _body,
        out_type=jax.ShapeDtypeStruct((b1n, D, b0n), jnp.float32),
        mesh=mesh,
        scratch_types=[
            pltpu.VMEM((b1n, CH), jnp.int32),
            pltpu.VMEM((b1n, CH), jnp.int32),
            pltpu.VMEM((CH, 2 * D), jnp.float32),
            pltpu.VMEM((CH, 2 * D), jnp.float32),
            pltpu.VMEM((D, CH + 1), jnp.float32),
            pltpu.VMEM((D, CH + 1), jnp.float32),
            pltpu.SemaphoreType.DMA,
            pltpu.SemaphoreType.DMA,
            pltpu.SemaphoreType.DMA,
            pltpu.SemaphoreType.DMA,
        ],
        compiler_params=pltpu.CompilerParams(
            use_tc_tiling_on_sc=True, needs_layout_passes=False),
    )(xT, w2)
    return jnp.transpose(outT, (2, 0, 1))


# speedup vs baseline: 1.7780x; 1.4883x over previous
import jax
import jax.numpy as jnp
from jax import lax
from jax.experimental import pallas as pl
from jax.experimental.pallas import tpu as pltpu
from jax.experimental.pallas import tpu_sc as plsc

NC = 2
NS = 16
NW = NC * NS
D = 64
CH = 128
L = 16


def _body(x_hbm, w_hbm, out_hbm, jv, ov, rawA, rawB, tbA, tbB, gA, gB, sA, sB):
    n_b1 = x_hbm.shape[0]
    wid = lax.axis_index("s") * NC + lax.axis_index("c")
    b0 = wid * CH

    pltpu.sync_copy(x_hbm.at[:, pl.ds(b0, CH)], jv)

    @pl.loop(0, n_b1)
    def _(b1):
        for g in range(CH // L):
            sl = pl.ds(g * L, L)
            v = jv[b1, sl]
            ov[b1, sl] = lax.shift_left(lax.bitwise_and(v, 1), 6)
            jv[b1, sl] = lax.shift_right_logical(v, 1)

    def fire_gather(b1, raw, sem):
        pltpu.async_copy(w_hbm.at[jv.at[b1]], raw, sem)

    def wait_gather(raw, sem):
        pltpu.make_async_copy(w_hbm.at[jv.at[0]], raw, sem).wait()

    tvec0 = lax.iota(jnp.int32, L)

    def transpose(b1, raw, tb):
        @pl.loop(0, CH // L)
        def _(g):
            tvec = tvec0 + g * L
            offv = ov[b1, pl.ds(g * L, L)]

            @pl.loop(0, D)
            def _(d):
                evec = lax.bitwise_and(tvec + d, D - 1)
                vals = plsc.load_gather(raw, [tvec, offv + evec])
                plsc.store_scatter(tb, [evec, tvec], vals)

    def fire_store(b1, tb, sem):
        pltpu.async_copy(
            tb.at[:, pl.ds(0, CH)], out_hbm.at[b1, :, pl.ds(b0, CH)], sem)

    def wait_store(tb, sem):
        pltpu.make_async_copy(
            tb.at[:, pl.ds(0, CH)], out_hbm.at[0, :, pl.ds(b0, CH)], sem
        ).wait()

    fire_gather(0, rawA, gA)

    @pl.loop(0, n_b1 // 2)
    def _(i):
        b1a = 2 * i
        b1b = 2 * i + 1
        wait_gather(rawA, gA)
        fire_gather(b1b, rawB, gB)

        @pl.when(i > 0)
        def _():
            wait_store(tbA, sA)

        transpose(b1a, rawA, tbA)
        fire_store(b1a, tbA, sA)

        wait_gather(rawB, gB)

        @pl.when(b1b + 1 < n_b1)
        def _():
            fire_gather(b1b + 1, rawA, gA)

        @pl.when(i > 0)
        def _():
            wait_store(tbB, sB)

        transpose(b1b, rawB, tbB)
        fire_store(b1b, tbB, sB)

    wait_store(tbA, sA)
    wait_store(tbB, sB)


def kernel(x, weight):
    b0n, b1n = x.shape
    nvoc, d = weight.shape
    assert d == D and b0n == NW * CH
    xT = x.T.astype(jnp.int32)
    w2 = weight.reshape(nvoc // 2, 2 * D)

    mesh = plsc.VectorSubcoreMesh(core_axis_name="c", subcore_axis_name="s")
    outT = pl.kernel(
        _body,
        out_type=jax.ShapeDtypeStruct((b1n, D, b0n), jnp.float32),
        mesh=mesh,
        scratch_types=[
            pltpu.VMEM((b1n, CH), jnp.int32),
            pltpu.VMEM((b1n, CH), jnp.int32),
            pltpu.VMEM((CH, 2 * D), jnp.float32),
            pltpu.VMEM((CH, 2 * D), jnp.float32),
            pltpu.VMEM((D, CH + 2), jnp.float32),
            pltpu.VMEM((D, CH + 2), jnp.float32),
            pltpu.SemaphoreType.DMA,
            pltpu.SemaphoreType.DMA,
            pltpu.SemaphoreType.DMA,
            pltpu.SemaphoreType.DMA,
        ],
        compiler_params=pltpu.CompilerParams(
            use_tc_tiling_on_sc=True, needs_layout_passes=False),
    )(xT, w2)
    return jnp.transpose(outT, (2, 0, 1))


# speedup vs baseline: 1.9133x; 1.0761x over previous
import jax
import jax.numpy as jnp
from jax import lax
from jax.experimental import pallas as pl
from jax.experimental.pallas import tpu as pltpu
from jax.experimental.pallas import tpu_sc as plsc

NC = 2
NS = 16
NW = NC * NS
D = 64
CH = 128
L = 16


def _body(x_hbm, w_hbm, out_hbm, jv, ov, rawA, rawB, tbA, tbB, gA, gB, sA, sB):
    n_b1 = x_hbm.shape[0]
    wid = lax.axis_index("s") * NC + lax.axis_index("c")
    b0 = wid * CH

    pltpu.sync_copy(x_hbm.at[:, pl.ds(b0, CH)], jv)

    @pl.loop(0, n_b1)
    def _(b1):
        for g in range(CH // L):
            sl = pl.ds(g * L, L)
            v = jv[b1, sl]
            ov[b1, sl] = lax.shift_left(lax.bitwise_and(v, 1), 6)
            jv[b1, sl] = lax.shift_right_logical(v, 1)

    def fire_gather(b1, raw, sem):
        pltpu.async_copy(w_hbm.at[jv.at[b1]], raw, sem)

    def wait_gather(raw, sem):
        pltpu.make_async_copy(w_hbm.at[jv.at[0]], raw, sem).wait()

    tvec0 = lax.iota(jnp.int32, L)

    def transpose(b1, raw, tb):
        @pl.loop(0, CH // L)
        def _(g):
            tvec = tvec0 + g * L
            offv = ov[b1, pl.ds(g * L, L)]

            @pl.loop(0, D, unroll=16)
            def _(d):
                evec = lax.bitwise_and(tvec + d, D - 1)
                vals = plsc.load_gather(raw, [tvec, offv + evec])
                plsc.store_scatter(tb, [evec, tvec], vals)

    def fire_store(b1, tb, sem):
        pltpu.async_copy(
            tb.at[:, pl.ds(0, CH)], out_hbm.at[b1, :, pl.ds(b0, CH)], sem)

    def wait_store(tb, sem):
        pltpu.make_async_copy(
            tb.at[:, pl.ds(0, CH)], out_hbm.at[0, :, pl.ds(b0, CH)], sem
        ).wait()

    fire_gather(0, rawA, gA)

    @pl.loop(0, n_b1 // 2)
    def _(i):
        b1a = 2 * i
        b1b = 2 * i + 1
        wait_gather(rawA, gA)
        fire_gather(b1b, rawB, gB)

        @pl.when(i > 0)
        def _():
            wait_store(tbA, sA)

        transpose(b1a, rawA, tbA)
        fire_store(b1a, tbA, sA)

        wait_gather(rawB, gB)

        @pl.when(b1b + 1 < n_b1)
        def _():
            fire_gather(b1b + 1, rawA, gA)

        @pl.when(i > 0)
        def _():
            wait_store(tbB, sB)

        transpose(b1b, rawB, tbB)
        fire_store(b1b, tbB, sB)

    wait_store(tbA, sA)
    wait_store(tbB, sB)


def kernel(x, weight):
    b0n, b1n = x.shape
    nvoc, d = weight.shape
    assert d == D and b0n == NW * CH
    xT = x.T.astype(jnp.int32)
    w2 = weight.reshape(nvoc // 2, 2 * D)

    mesh = plsc.VectorSubcoreMesh(core_axis_name="c", subcore_axis_name="s")
    outT = pl.kernel(
        _body,
        out_type=jax.ShapeDtypeStruct((b1n, D, b0n), jnp.float32),
        mesh=mesh,
        scratch_types=[
            pltpu.VMEM((b1n, CH), jnp.int32),
            pltpu.VMEM((b1n, CH), jnp.int32),
            pltpu.VMEM((CH, 2 * D), jnp.float32),
            pltpu.VMEM((CH, 2 * D), jnp.float32),
            pltpu.VMEM((D, CH + 2), jnp.float32),
            pltpu.VMEM((D, CH + 2), jnp.float32),
            pltpu.SemaphoreType.DMA,
            pltpu.SemaphoreType.DMA,
            pltpu.SemaphoreType.DMA,
            pltpu.SemaphoreType.DMA,
        ],
        compiler_params=pltpu.CompilerParams(
            use_tc_tiling_on_sc=True, needs_layout_passes=False),
    )(xT, w2)
    return jnp.transpose(outT, (2, 0, 1))
